# baseline (speedup 1.0000x reference)
import functools

import numpy as np
import jax
import jax.numpy as jnp
from jax import lax
from jax.experimental import pallas as pl
from jax.experimental.pallas import tpu as pltpu
from jax.experimental.pallas import tpu_sc as plsc

VOCAB = 100000
D = 128
B = 16
TX = 512
TY = 2048
T_OUT = TX + TY

NC = 2
NS = 16
NW = NC * NS
ROWS_W = (B * TY) // NW
CHUNK = 128
NCHUNK = ROWS_W // CHUNK
NSLOT = 5
XROWS_W = (B * TX) // NW


def _sine_pe(length, dim):
    pos = np.arange(length, dtype=np.float32)[:, None]
    div = np.exp(np.arange(0, dim, 2, dtype=np.float32) * -(np.log(10000.0) / dim))
    pe = np.zeros((length, dim), dtype=np.float32)
    pe[:, 0::2] = np.sin(pos * div)
    pe[:, 1::2] = np.cos(pos * div)
    return pe


_PE = _sine_pe(TY, D)

_mesh = plsc.VectorSubcoreMesh(core_axis_name="c", subcore_axis_name="s")


@functools.partial(
    pl.kernel,
    out_type=jax.ShapeDtypeStruct((B, T_OUT, D), jnp.float32),
    mesh=_mesh,
    scratch_types=[
        pltpu.VMEM((NCHUNK, CHUNK), jnp.int32),
        pltpu.VMEM((NSLOT, CHUNK, D), jnp.float32),
        pltpu.VMEM_SHARED((TY // 2, D), jnp.float32),
        [pltpu.SemaphoreType.DMA] * NSLOT,
        [pltpu.SemaphoreType.DMA] * NSLOT,
        [pltpu.SemaphoreType.DMA] * NSLOT,
        pltpu.SemaphoreType.DMA,
        pltpu.SemaphoreType.DMA,
        pltpu.SemaphoreType.DMA,
    ],
)
def _emb_kernel(x_hbm, y_hbm, table_hbm, ape_hbm, out_hbm,
                idx_v, rows_v, ape_sh, psems, gsems, osems, xsem, isem, ssem):
    s = lax.axis_index("s")
    c = lax.axis_index("c")
    b = s
    half = c
    t0 = half * (TY // 2)

    stage_rows = (TY // 2) // NS
    stage_cp = pltpu.make_async_copy(
        ape_hbm.at[pl.ds(t0 + s * stage_rows, stage_rows)],
        ape_sh.at[pl.ds(s * stage_rows, stage_rows)], ssem)
    stage_cp.start()

    icp = pltpu.make_async_copy(
        y_hbm.at[pl.ds((b * NC + half) * NCHUNK, NCHUNK)], idx_v, isem)
    icp.start()
    xcp = pltpu.make_async_copy(
        x_hbm.at[b, pl.ds(half * XROWS_W, XROWS_W)],
        out_hbm.at[b, pl.ds(half * XROWS_W, XROWS_W)], xsem)
    xcp.start()

    stage_cp.wait()
    plsc.subcore_barrier()

    def pe_start(cch, slot):
        return pltpu.async_copy(
            ape_sh.at[pl.ds(cch * CHUNK, CHUNK)], rows_v.at[slot],
            psems[slot])

    def gather_start(cch, slot):
        return pltpu.async_copy(
            table_hbm.at[idx_v.at[cch]], rows_v.at[slot], gsems[slot],
            add=True)

    def out_start(cch, slot):
        return pltpu.async_copy(
            rows_v.at[slot],
            out_hbm.at[b, pl.ds(TX + t0 + cch * CHUNK, CHUNK)], osems[slot])

    icp.wait()

    p_cps = [None] * NSLOT
    g_cps = [None] * NSLOT
    o_cps = [None] * NSLOT
    for step in range(NCHUNK + 2):
        c_p = step
        c_g = step - 1
        c_o = step - 2
        if c_p < NCHUNK:
            sp = c_p % NSLOT
            if o_cps[sp] is not None:
                o_cps[sp].wait()
                o_cps[sp] = None
            p_cps[sp] = pe_start(c_p, sp)
        if 0 <= c_g < NCHUNK:
            sg = c_g % NSLOT
            p_cps[sg].wait()
            g_cps[sg] = gather_start(c_g, sg)
        if 0 <= c_o < NCHUNK:
            so = c_o % NSLOT
            g_cps[so].wait()
            o_cps[so] = out_start(c_o, so)

    xcp.wait()
    for cp in o_cps:
        if cp is not None:
            cp.wait()


def kernel(x, y, table, alpha):
    y2 = y.astype(jnp.int32).reshape(NW * NCHUNK, CHUNK)
    ape = alpha * jnp.asarray(_PE)
    return _emb_kernel(x, y2, table, ape)


# speedup vs baseline: 3.8682x; 3.8682x over previous
import functools

import numpy as np
import jax
import jax.numpy as jnp
from jax import lax
from jax.experimental import pallas as pl
from jax.experimental.pallas import tpu as pltpu
from jax.experimental.pallas import tpu_sc as plsc

VOCAB = 100000
D = 128
B = 16
TX = 512
TY = 2048
T_OUT = TX + TY

NC = 2
NS = 16
NW = NC * NS
ROWS_W = (B * TY) // NW
CHUNK = 128
NCHUNK = ROWS_W // CHUNK
NSLOT = 5
XROWS_W = (B * TX) // NW


def _sine_pe(length, dim):
    pos = np.arange(length, dtype=np.float32)[:, None]
    div = np.exp(np.arange(0, dim, 2, dtype=np.float32) * -(np.log(10000.0) / dim))
    pe = np.zeros((length, dim), dtype=np.float32)
    pe[:, 0::2] = np.sin(pos * div)
    pe[:, 1::2] = np.cos(pos * div)
    return pe


_PE = _sine_pe(TY, D)

_mesh = plsc.VectorSubcoreMesh(core_axis_name="c", subcore_axis_name="s")


@functools.partial(
    pl.kernel,
    out_type=jax.ShapeDtypeStruct((B, T_OUT, D), jnp.float32),
    mesh=_mesh,
    scratch_types=[
        pltpu.VMEM((NCHUNK, CHUNK), jnp.int32),
        pltpu.VMEM((NSLOT, CHUNK, D), jnp.float32),
        pltpu.VMEM((XROWS_W, D), jnp.float32),
        pltpu.VMEM_SHARED((TY // 2, D), jnp.float32),
        [pltpu.SemaphoreType.DMA] * NSLOT,
        [pltpu.SemaphoreType.DMA] * NSLOT,
        [pltpu.SemaphoreType.DMA] * NSLOT,
        pltpu.SemaphoreType.DMA,
        pltpu.SemaphoreType.DMA,
        pltpu.SemaphoreType.DMA,
    ],
)
def _emb_kernel(x_hbm, y_hbm, table_hbm, ape_hbm, out_hbm,
                idx_v, rows_v, x_v, ape_sh, psems, gsems, osems, xsem, isem,
                ssem):
    s = lax.axis_index("s")
    c = lax.axis_index("c")
    b = s
    half = c
    t0 = half * (TY // 2)

    stage_rows = (TY // 2) // NS
    stage_cp = pltpu.make_async_copy(
        ape_hbm.at[pl.ds(t0 + s * stage_rows, stage_rows)],
        ape_sh.at[pl.ds(s * stage_rows, stage_rows)], ssem)
    stage_cp.start()

    icp = pltpu.make_async_copy(
        y_hbm.at[pl.ds((b * NC + half) * NCHUNK, NCHUNK)], idx_v, isem)
    icp.start()
    xin = pltpu.make_async_copy(
        x_hbm.at[b, pl.ds(half * XROWS_W, XROWS_W)], x_v, xsem)
    xin.start()

    stage_cp.wait()
    plsc.subcore_barrier()

    def pe_start(cch, slot):
        return pltpu.async_copy(
            ape_sh.at[pl.ds(cch * CHUNK, CHUNK)], rows_v.at[slot],
            psems[slot])

    def gather_start(cch, slot):
        return pltpu.async_copy(
            table_hbm.at[idx_v.at[cch]], rows_v.at[slot], gsems[slot],
            add=True)

    def out_start(cch, slot):
        return pltpu.async_copy(
            rows_v.at[slot],
            out_hbm.at[b, pl.ds(TX + t0 + cch * CHUNK, CHUNK)], osems[slot])

    icp.wait()
    xin.wait()
    xout = pltpu.make_async_copy(
        x_v, out_hbm.at[b, pl.ds(half * XROWS_W, XROWS_W)], xsem)
    xout.start()

    p_cps = [None] * NSLOT
    g_cps = [None] * NSLOT
    o_cps = [None] * NSLOT
    for step in range(NCHUNK + 2):
        c_p = step
        c_g = step - 1
        c_o = step - 2
        if c_p < NCHUNK:
            sp = c_p % NSLOT
            if o_cps[sp] is not None:
                o_cps[sp].wait()
                o_cps[sp] = None
            p_cps[sp] = pe_start(c_p, sp)
        if 0 <= c_g < NCHUNK:
            sg = c_g % NSLOT
            p_cps[sg].wait()
            g_cps[sg] = gather_start(c_g, sg)
        if 0 <= c_o < NCHUNK:
            so = c_o % NSLOT
            g_cps[so].wait()
            o_cps[so] = out_start(c_o, so)

    xout.wait()
    for cp in o_cps:
        if cp is not None:
            cp.wait()


def kernel(x, y, table, alpha):
    y2 = y.astype(jnp.int32).reshape(NW * NCHUNK, CHUNK)
    ape = alpha * jnp.asarray(_PE)
    return _emb_kernel(x, y2, table, ape)


# speedup vs baseline: 3.9203x; 1.0135x over previous
import functools

import numpy as np
import jax
import jax.numpy as jnp
from jax import lax
from jax.experimental import pallas as pl
from jax.experimental.pallas import tpu as pltpu
from jax.experimental.pallas import tpu_sc as plsc

VOCAB = 100000
D = 128
B = 16
TX = 512
TY = 2048
T_OUT = TX + TY

NC = 2
NS = 16
NW = NC * NS
W_T = TY // NW
NSLOT = 4
LOOKAHEAD = 2
XROWS_W = (B * TX) // NW
LANES = 16


def _sine_pe(length, dim):
    pos = np.arange(length, dtype=np.float32)[:, None]
    div = np.exp(np.arange(0, dim, 2, dtype=np.float32) * -(np.log(10000.0) / dim))
    pe = np.zeros((length, dim), dtype=np.float32)
    pe[:, 0::2] = np.sin(pos * div)
    pe[:, 1::2] = np.cos(pos * div)
    return pe


_PE = _sine_pe(TY, D)

_mesh = plsc.VectorSubcoreMesh(core_axis_name="c", subcore_axis_name="s")


@functools.partial(
    pl.kernel,
    out_type=jax.ShapeDtypeStruct((B, T_OUT, D), jnp.float32),
    mesh=_mesh,
    scratch_types=[
        pltpu.VMEM((B, W_T), jnp.int32),
        pltpu.VMEM((NSLOT, W_T, D), jnp.float32),
        pltpu.VMEM((W_T, D), jnp.float32),
        pltpu.VMEM((XROWS_W, D), jnp.float32),
        [pltpu.SemaphoreType.DMA] * NSLOT,
        [pltpu.SemaphoreType.DMA] * NSLOT,
        pltpu.SemaphoreType.DMA,
        pltpu.SemaphoreType.DMA,
        pltpu.SemaphoreType.DMA,
    ],
)
def _emb_kernel(x_hbm, y_hbm, table_hbm, ape_hbm, out_hbm,
                idx_v, rows_v, pe_v, x_v, gsems, osems, xsem, isem, psem):
    s = lax.axis_index("s")
    c = lax.axis_index("c")
    w = s * NC + c
    tw = w * W_T

    icps = []
    for bb in range(B):
        cp = pltpu.make_async_copy(
            y_hbm.at[bb, pl.ds(tw, W_T)], idx_v.at[bb], isem)
        cp.start()
        icps.append(cp)
    pcp = pltpu.make_async_copy(ape_hbm.at[pl.ds(tw, W_T)], pe_v, psem)
    pcp.start()
    xb = s
    xhalf = c
    xin = pltpu.make_async_copy(
        x_hbm.at[xb, pl.ds(xhalf * XROWS_W, XROWS_W)], x_v, xsem)
    xin.start()

    for cp in icps:
        cp.wait()

    xin.wait()
    xout = pltpu.make_async_copy(
        x_v, out_hbm.at[xb, pl.ds(xhalf * XROWS_W, XROWS_W)], xsem)
    xout.start()

    pcp.wait()

    def gather_start(bb, slot):
        return pltpu.async_copy(
            table_hbm.at[idx_v.at[bb]], rows_v.at[slot], gsems[slot])

    def out_start(bb, slot):
        return pltpu.async_copy(
            rows_v.at[slot], out_hbm.at[bb, pl.ds(TX + tw, W_T)], osems[slot])

    g_cps = [None] * NSLOT
    o_cps = [None] * NSLOT
    for step in range(B + LOOKAHEAD):
        c_g = step
        c_a = step - LOOKAHEAD
        if c_g < B:
            sg = c_g % NSLOT
            if o_cps[sg] is not None:
                o_cps[sg].wait()
                o_cps[sg] = None
            g_cps[sg] = gather_start(c_g, sg)
        if 0 <= c_a < B:
            sa = c_a % NSLOT
            g_cps[sa].wait()

            def add_body(r, carry):
                for j in range(D // LANES):
                    sl = pl.ds(j * LANES, LANES)
                    plsc.addupdate(rows_v.at[sa, r, sl], pe_v[r, sl])
                return carry

            lax.fori_loop(0, W_T, add_body, 0)
            o_cps[sa] = out_start(c_a, sa)

    xout.wait()
    for cp in o_cps:
        if cp is not None:
            cp.wait()


def kernel(x, y, table, alpha):
    ape = alpha * jnp.asarray(_PE)
    return _emb_kernel(x, y.astype(jnp.int32), table, ape)
